# baseline (speedup 1.0000x reference)
import functools

import jax
import jax.numpy as jnp
from jax import lax
from jax.experimental import pallas as pl
from jax.experimental.pallas import tpu as pltpu
from jax.experimental.pallas import tpu_sc as plsc

V = 100001
VP = 100008
D = 64
B = 16384
NC = 2
NS = 16
NW = NC * NS
B_PER_W = B // NW
CHUNK = 128
NCHUNK = B_PER_W // CHUNK
K1_BLK = 512

_mesh = plsc.VectorSubcoreMesh(core_axis_name="c", subcore_axis_name="s")


def _k1_body(tabT_ref, t2_ref):
    x = tabT_ref[...]
    eye = jnp.eye(D, dtype=jnp.float32)
    y = jax.lax.dot_general(
        x, eye, (((0,), (0,)), ((), ())),
        preferred_element_type=jnp.float32)
    t2_ref[...] = jnp.concatenate(
        [y, jnp.zeros((K1_BLK, 64), jnp.float32)], axis=1)


def _k1(tabT):
    grid = (VP // K1_BLK + (1 if VP % K1_BLK else 0),)
    return pl.pallas_call(
        _k1_body,
        grid=grid,
        in_specs=[pl.BlockSpec((D, K1_BLK), lambda j: (0, j))],
        out_specs=pl.BlockSpec((K1_BLK, 128), lambda j: (j, 0)),
        out_shape=jax.ShapeDtypeStruct((VP, 128), jnp.float32),
    )(tabT)


@functools.partial(
    pl.kernel,
    mesh=_mesh,
    out_type=jax.ShapeDtypeStruct((B, 128), jnp.float32),
    scratch_types=[
        pltpu.VMEM((B_PER_W,), jnp.int32),
        pltpu.VMEM((B_PER_W, 128), jnp.float32),
        pltpu.SemaphoreType.DMA,
    ],
    compiler_params=pltpu.CompilerParams(
        use_tc_tiling_on_sc=True, needs_layout_passes=False),
)
def _k2(t2, idx_hbm, g_out, idx_v, rows_v, sem):
    wid = lax.axis_index("s") * NC + lax.axis_index("c")
    base = wid * B_PER_W
    pltpu.sync_copy(idx_hbm.at[pl.ds(base, B_PER_W)], idx_v)
    copies = []
    for m in range(NCHUNK):
        copies.append(
            pltpu.async_copy(
                t2.at[idx_v.at[pl.ds(m * CHUNK, CHUNK)]],
                rows_v.at[pl.ds(m * CHUNK, CHUNK), :],
                sem,
            )
        )
    for c in copies:
        c.wait()
    pltpu.sync_copy(rows_v, g_out.at[pl.ds(base, B_PER_W), :])


def _k3_body(g_ref, outT_ref):
    x = g_ref[...]
    outT_ref[...] = x[:, :D].T


def _k3(g):
    return pl.pallas_call(
        _k3_body,
        grid=(B // K1_BLK,),
        in_specs=[pl.BlockSpec((K1_BLK, 128), lambda j: (j, 0))],
        out_specs=pl.BlockSpec((D, K1_BLK), lambda j: (0, j)),
        out_shape=jax.ShapeDtypeStruct((D, B), jnp.float32),
    )(g)


def kernel(layer_index, table):
    t2 = _k1(table.T)
    g = _k2(t2, layer_index.astype(jnp.int32))
    return _k3(g).T


# speedup vs baseline: 2.1812x; 2.1812x over previous
import functools

import jax
import jax.numpy as jnp
from jax import lax
from jax.experimental import pallas as pl
from jax.experimental.pallas import tpu as pltpu
from jax.experimental.pallas import tpu_sc as plsc

V = 100001
VP = 100008
D = 64
B = 16384
NC = 2
NS = 16
NW = NC * NS
B_PER_W = B // NW
CHUNK = 128
NCHUNK = B_PER_W // CHUNK
K1_BLK = 4096
K3_BLK = 512

_mesh = plsc.VectorSubcoreMesh(core_axis_name="c", subcore_axis_name="s")


def _k1_body(tabT_ref, t2_ref):
    x = tabT_ref[...]
    t2_ref[...] = jnp.concatenate(
        [x.T, jnp.zeros((K1_BLK, 64), jnp.float32)], axis=1)


def _k1(tabT):
    grid = (VP // K1_BLK + (1 if VP % K1_BLK else 0),)
    return pl.pallas_call(
        _k1_body,
        grid=grid,
        in_specs=[pl.BlockSpec((D, K1_BLK), lambda j: (0, j))],
        out_specs=pl.BlockSpec((K1_BLK, 128), lambda j: (j, 0)),
        out_shape=jax.ShapeDtypeStruct((VP, 128), jnp.float32),
    )(tabT)


@functools.partial(
    pl.kernel,
    mesh=_mesh,
    out_type=jax.ShapeDtypeStruct((B, 128), jnp.float32),
    scratch_types=[
        pltpu.VMEM((B_PER_W,), jnp.int32),
        pltpu.VMEM((B_PER_W, 128), jnp.float32),
        pltpu.SemaphoreType.DMA,
    ],
    compiler_params=pltpu.CompilerParams(
        use_tc_tiling_on_sc=True, needs_layout_passes=False),
)
def _k2(t2, idx_hbm, g_out, idx_v, rows_v, sem):
    wid = lax.axis_index("s") * NC + lax.axis_index("c")
    base = wid * B_PER_W
    pltpu.sync_copy(idx_hbm.at[pl.ds(base, B_PER_W)], idx_v)
    copies = []
    for m in range(NCHUNK):
        copies.append(
            pltpu.async_copy(
                t2.at[idx_v.at[pl.ds(m * CHUNK, CHUNK)]],
                rows_v.at[pl.ds(m * CHUNK, CHUNK), :],
                sem,
            )
        )
    for c in copies:
        c.wait()
    pltpu.sync_copy(rows_v, g_out.at[pl.ds(base, B_PER_W), :])


def _k3_body(g_ref, outT_ref):
    x = g_ref[...]
    outT_ref[...] = x[:, :D].T


def _k3(g):
    return pl.pallas_call(
        _k3_body,
        grid=(B // K3_BLK,),
        in_specs=[pl.BlockSpec((K3_BLK, 128), lambda j: (j, 0))],
        out_specs=pl.BlockSpec((D, K3_BLK), lambda j: (0, j)),
        out_shape=jax.ShapeDtypeStruct((D, B), jnp.float32),
    )(g)


def kernel(layer_index, table):
    t2 = _k1(table.T)
    g = _k2(t2, layer_index.astype(jnp.int32))
    return _k3(g).T


# speedup vs baseline: 2.9973x; 1.3741x over previous
import functools

import jax
import jax.numpy as jnp
from jax import lax
from jax.experimental import pallas as pl
from jax.experimental.pallas import tpu as pltpu
from jax.experimental.pallas import tpu_sc as plsc

V = 100001
VP = 100008
D = 64
B = 16384
NC = 2
NS = 16
NW = NC * NS
B_PER_W = B // NW
CHUNK = 128
NCHUNK = B_PER_W // CHUNK
K1_BLK = 8192
K3_BLK = 4096

_mesh = plsc.VectorSubcoreMesh(core_axis_name="c", subcore_axis_name="s")


def _k1_body(tabT_ref, t2_ref):
    x = tabT_ref[...]
    t2_ref[...] = jnp.concatenate(
        [x.T, jnp.zeros((K1_BLK, 64), jnp.float32)], axis=1)


def _k1(tabT):
    grid = (VP // K1_BLK + (1 if VP % K1_BLK else 0),)
    return pl.pallas_call(
        _k1_body,
        grid=grid,
        in_specs=[pl.BlockSpec((D, K1_BLK), lambda j: (0, j))],
        out_specs=pl.BlockSpec((K1_BLK, 128), lambda j: (j, 0)),
        out_shape=jax.ShapeDtypeStruct((VP, 128), jnp.float32),
    )(tabT)


@functools.partial(
    pl.kernel,
    mesh=_mesh,
    out_type=jax.ShapeDtypeStruct((B, 128), jnp.float32),
    scratch_types=[
        pltpu.VMEM((B_PER_W,), jnp.int32),
        pltpu.VMEM((B_PER_W, 128), jnp.float32),
        pltpu.SemaphoreType.DMA,
    ],
    compiler_params=pltpu.CompilerParams(
        use_tc_tiling_on_sc=True, needs_layout_passes=False),
)
def _k2(t2, idx_hbm, g_out, idx_v, rows_v, sem):
    wid = lax.axis_index("s") * NC + lax.axis_index("c")
    base = wid * B_PER_W
    pltpu.sync_copy(idx_hbm.at[pl.ds(base, B_PER_W)], idx_v)
    copies = []
    for m in range(NCHUNK):
        copies.append(
            pltpu.async_copy(
                t2.at[idx_v.at[pl.ds(m * CHUNK, CHUNK)]],
                rows_v.at[pl.ds(m * CHUNK, CHUNK), :],
                sem,
            )
        )
    for c in copies:
        c.wait()
    pltpu.sync_copy(rows_v, g_out.at[pl.ds(base, B_PER_W), :])


def _k3_body(g_ref, outT_ref):
    x = g_ref[...]
    outT_ref[...] = x[:, :D].T


def _k3(g):
    return pl.pallas_call(
        _k3_body,
        grid=(B // K3_BLK,),
        in_specs=[pl.BlockSpec((K3_BLK, 128), lambda j: (j, 0))],
        out_specs=pl.BlockSpec((D, K3_BLK), lambda j: (0, j)),
        out_shape=jax.ShapeDtypeStruct((D, B), jnp.float32),
    )(g)


def kernel(layer_index, table):
    t2 = _k1(table.T)
    g = _k2(t2, layer_index.astype(jnp.int32))
    return _k3(g).T
